# baseline (speedup 1.0000x reference)
import functools

import jax
import jax.numpy as jnp
from jax import lax
from jax.experimental import pallas as pl
from jax.experimental.pallas import tpu as pltpu
from jax.experimental.pallas import tpu_sc as plsc

N = 16384
VOCAB = 1000
HIDDEN = 128

BC = 2048

NC, NS = 2, 16
NW = NC * NS
BPW = N // NW


def _argmax_body(xt_ref, idx_ref):
    x = xt_ref[...]
    m0 = jnp.max(x, axis=0, keepdims=True)
    row = lax.broadcasted_iota(jnp.int32, x.shape, 0)
    cand = jnp.where(x == m0, row, VOCAB)
    idx_ref[...] = jnp.min(cand, axis=0)


def _tc_argmax(clique_attr_t):
    return pl.pallas_call(
        _argmax_body,
        grid=(N // BC,),
        in_specs=[pl.BlockSpec((VOCAB, BC), lambda i: (0, i))],
        out_specs=pl.BlockSpec((BC,), lambda i: (i,)),
        out_shape=jax.ShapeDtypeStruct((N,), jnp.int32),
    )(clique_attr_t)


NSUB = 4
SUB = BPW // NSUB


def _sc_gather_body(table_hbm, idx_hbm, out_hbm,
                    iv0, iv1, iv2, iv3, rows_v, g0, g1, g2, g3):
    ivs = (iv0, iv1, iv2, iv3)
    sems = (g0, g1, g2, g3)
    wid = lax.axis_index("s") * NC + lax.axis_index("c")
    base = wid * BPW
    for s in range(NSUB):
        pltpu.sync_copy(idx_hbm.at[pl.ds(base + s * SUB, SUB)], ivs[s])
    for s in range(NSUB):
        pltpu.make_async_copy(
            table_hbm.at[ivs[s]],
            rows_v.at[pl.ds(s * SUB, SUB), :], sems[s]).start()
    for s in range(NSUB):
        pltpu.make_async_copy(
            table_hbm.at[ivs[s]],
            rows_v.at[pl.ds(s * SUB, SUB), :], sems[s]).wait()
        pltpu.sync_copy(rows_v.at[pl.ds(s * SUB, SUB), :],
                        out_hbm.at[pl.ds(base + s * SUB, SUB)])


@functools.cache
def _make_sc_gather():
    mesh = plsc.VectorSubcoreMesh(
        core_axis_name="c", subcore_axis_name="s", num_cores=NC, num_subcores=NS
    )
    return pl.kernel(
        _sc_gather_body,
        out_type=jax.ShapeDtypeStruct((N, HIDDEN), jnp.float32),
        mesh=mesh,
        scratch_types=[
            pltpu.VMEM((SUB,), jnp.int32),
            pltpu.VMEM((SUB,), jnp.int32),
            pltpu.VMEM((SUB,), jnp.int32),
            pltpu.VMEM((SUB,), jnp.int32),
            pltpu.VMEM((BPW, HIDDEN), jnp.float32),
            pltpu.SemaphoreType.DMA,
            pltpu.SemaphoreType.DMA,
            pltpu.SemaphoreType.DMA,
            pltpu.SemaphoreType.DMA,
        ],
    )


@jax.jit
def kernel(clique_attr, embedding_weight):
    idx = _tc_argmax(clique_attr.T)
    return _make_sc_gather()(embedding_weight, idx)


# speedup vs baseline: 1.0295x; 1.0295x over previous
import functools

import jax
import jax.numpy as jnp
from jax import lax
from jax.experimental import pallas as pl
from jax.experimental.pallas import tpu as pltpu
from jax.experimental.pallas import tpu_sc as plsc

N = 16384
VOCAB = 1000
HIDDEN = 128

BC = 4096

NC, NS = 2, 16
NW = NC * NS
BPW = N // NW


def _argmax_body(xt_ref, idx_ref):
    x = xt_ref[...]
    m0 = jnp.max(x, axis=0, keepdims=True)
    row = lax.broadcasted_iota(jnp.int32, x.shape, 0)
    cand = jnp.where(x == m0, row, VOCAB)
    idx_ref[...] = jnp.min(cand, axis=0)


def _tc_argmax(clique_attr_t):
    return pl.pallas_call(
        _argmax_body,
        grid=(N // BC,),
        in_specs=[pl.BlockSpec((VOCAB, BC), lambda i: (0, i))],
        out_specs=pl.BlockSpec((BC,), lambda i: (i,)),
        out_shape=jax.ShapeDtypeStruct((N,), jnp.int32),
    )(clique_attr_t)


def _sc_gather_body(table_hbm, idx_hbm, out_hbm, idx_v, rows_v, gsem):
    wid = lax.axis_index("s") * NC + lax.axis_index("c")
    base = wid * BPW
    pltpu.sync_copy(idx_hbm.at[pl.ds(base, BPW)], idx_v)
    pltpu.async_copy(table_hbm.at[idx_v], rows_v, gsem).wait()
    pltpu.sync_copy(rows_v, out_hbm.at[pl.ds(base, BPW)])


@functools.cache
def _make_sc_gather():
    mesh = plsc.VectorSubcoreMesh(
        core_axis_name="c", subcore_axis_name="s", num_cores=NC, num_subcores=NS
    )
    return pl.kernel(
        _sc_gather_body,
        out_type=jax.ShapeDtypeStruct((N, HIDDEN), jnp.float32),
        mesh=mesh,
        scratch_types=[
            pltpu.VMEM((BPW,), jnp.int32),
            pltpu.VMEM((BPW, HIDDEN), jnp.float32),
            pltpu.SemaphoreType.DMA,
        ],
    )


@jax.jit
def kernel(clique_attr, embedding_weight):
    idx = _tc_argmax(clique_attr.T)
    return _make_sc_gather()(embedding_weight, idx)


# speedup vs baseline: 1.0457x; 1.0157x over previous
import functools

import jax
import jax.numpy as jnp
from jax import lax
from jax.experimental import pallas as pl
from jax.experimental.pallas import tpu as pltpu
from jax.experimental.pallas import tpu_sc as plsc

N = 16384
VOCAB = 1000
HIDDEN = 128

BC = 2048

NC, NS = 2, 16
NW = NC * NS
BPW = N // NW


def _argmax_body(xt_ref, idx_ref):
    x = xt_ref[...]
    m0 = jnp.max(x, axis=0, keepdims=True)
    row = lax.broadcasted_iota(jnp.int32, x.shape, 0)
    cand = jnp.where(x == m0, row, VOCAB)
    idx_ref[...] = jnp.min(cand, axis=0)


def _tc_argmax(clique_attr_t):
    return pl.pallas_call(
        _argmax_body,
        grid=(N // BC,),
        in_specs=[pl.BlockSpec((VOCAB, BC), lambda i: (0, i))],
        out_specs=pl.BlockSpec((BC,), lambda i: (i,)),
        out_shape=jax.ShapeDtypeStruct((N,), jnp.int32),
    )(clique_attr_t)


def _sc_gather_body(table_hbm, idx_hbm, out_hbm, idx_v, rows_v, gsem):
    wid = lax.axis_index("s") * NC + lax.axis_index("c")
    base = wid * BPW
    pltpu.sync_copy(idx_hbm.at[pl.ds(base, BPW)], idx_v)
    pltpu.async_copy(table_hbm.at[idx_v], rows_v, gsem).wait()
    pltpu.sync_copy(rows_v, out_hbm.at[pl.ds(base, BPW)])


@functools.cache
def _make_sc_gather():
    mesh = plsc.VectorSubcoreMesh(
        core_axis_name="c", subcore_axis_name="s", num_cores=NC, num_subcores=NS
    )
    return pl.kernel(
        _sc_gather_body,
        out_type=jax.ShapeDtypeStruct((N, HIDDEN), jnp.float32),
        mesh=mesh,
        scratch_types=[
            pltpu.VMEM((BPW,), jnp.int32),
            pltpu.VMEM((BPW, HIDDEN), jnp.float32),
            pltpu.SemaphoreType.DMA,
        ],
        compiler_params=pltpu.CompilerParams(skip_device_barrier=True),
    )


@jax.jit
def kernel(clique_attr, embedding_weight):
    idx = _tc_argmax(clique_attr.T)
    return _make_sc_gather()(embedding_weight, idx)
